# baseline (speedup 1.0000x reference)
import functools

import jax
import jax.numpy as jnp
from jax import lax
from jax.experimental import pallas as pl
from jax.experimental.pallas import tpu as pltpu
from jax.experimental.pallas import tpu_sc as plsc

_ROWS = 128
_COLS = 32768
_K = 16384
_ROW_BLOCK = 16

_NC = 2
_NS = 16
_NW = _NC * _NS
_RPW = _ROWS // _NW
_NB = 256
_CAP = 4096


def _keys_body(z_ref, eps_ref, keys_ref):
    eps = jnp.clip(eps_ref[...], 1e-6, 1.0 - 1e-6)
    s = z_ref[...] - jnp.log(-jnp.log(eps))
    b = lax.bitcast_convert_type(s, jnp.int32)
    keys_ref[...] = b ^ ((b >> 31) & jnp.int32(0x7FFFFFFF))


def _tc_keys(z, eps):
    spec = pl.BlockSpec((_ROW_BLOCK, _COLS), lambda i: (i, 0))
    return pl.pallas_call(
        _keys_body,
        grid=(_ROWS // _ROW_BLOCK,),
        in_specs=[spec, spec],
        out_specs=spec,
        out_shape=jax.ShapeDtypeStruct((_ROWS, _COLS), jnp.int32),
    )(z, eps)


_mesh = plsc.VectorSubcoreMesh(core_axis_name="c", subcore_axis_name="s")


@functools.partial(
    pl.kernel,
    out_type=jax.ShapeDtypeStruct((_ROWS, _COLS), jnp.float32),
    mesh=_mesh,
    scratch_types=[
        pltpu.VMEM((_COLS,), jnp.int32),
        pltpu.VMEM((_COLS,), jnp.float32),
        pltpu.VMEM((_NB,), jnp.int32),
        pltpu.VMEM((_NB,), jnp.int32),
        pltpu.VMEM((_CAP,), jnp.int32),
    ],
    compiler_params=pltpu.CompilerParams(needs_layout_passes=False),
)
def _sc_select(keys_hbm, out_hbm, keys_v, out_v, h1_v, h2_v, surv_v):
    wid = lax.axis_index("s") * _NC + lax.axis_index("c")
    nchunk = _COLS // 16
    zeros16 = jnp.zeros((16,), jnp.int32)
    ones16 = jnp.ones((16,), jnp.int32)
    lanes = lax.iota(jnp.int32, 16)

    def scan_hist(h_ref, kk):
        def sbody(i, carry):
            cnt_ge, cnt_ab, tot = carry
            c = (_NB // 16 - 1) - i
            t16 = h_ref[pl.ds(c * 16, 16)]
            t_rev = lax.rev(t16, (0,))
            s_rev = plsc.cumsum(t_rev) + tot
            ge = s_rev >= kk
            cnt_ge = cnt_ge + jnp.sum(ge.astype(jnp.int32))
            cnt_ab = cnt_ab + jnp.sum(jnp.where(ge, 0, t_rev))
            tot = tot + jnp.sum(t16)
            return cnt_ge, cnt_ab, tot
        z = jnp.int32(0)
        cnt_ge, cnt_ab, _ = lax.fori_loop(0, _NB // 16, sbody, (z, z, z))
        return cnt_ge - 1, cnt_ab

    def row_body(r, carry):
        row = wid * _RPW + r
        pltpu.sync_copy(keys_hbm.at[row], keys_v)

        def zero_body(i, c):
            h1_v[pl.ds(i * 16, 16)] = zeros16
            h2_v[pl.ds(i * 16, 16)] = zeros16
            return c
        lax.fori_loop(0, _NB // 16, zero_body, jnp.int32(0))

        def p1(i, c):
            v = keys_v[pl.ds(i * 16, 16)]
            plsc.addupdate_scatter(h1_v, [(v >> 24) + 128], ones16)
            return c
        lax.fori_loop(0, nchunk, p1, jnp.int32(0))
        b1, ca1 = scan_hist(h1_v, _K)
        k2 = _K - ca1

        def p2(i, c):
            v = keys_v[pl.ds(i * 16, 16)]
            pred = ((v >> 24) + 128) == b1
            plsc.addupdate_scatter(h2_v, [(v >> 16) & 0xFF], ones16, mask=pred)
            return c
        lax.fori_loop(0, nchunk, p2, jnp.int32(0))
        b2, ca2 = scan_hist(h2_v, k2)
        k3 = k2 - ca2
        t_hi = ((b1 - 128) << 8) | b2

        def p3(i, cnt):
            v = keys_v[pl.ds(i * 16, 16)]
            pred = (v >> 16) == t_hi
            off = jnp.minimum(cnt, _CAP - 16)
            plsc.store_compressed(surv_v.at[pl.ds(off, 16)], v, mask=pred)
            return cnt + jnp.sum(pred.astype(jnp.int32))
        nsurv = lax.fori_loop(0, nchunk, p3, jnp.int32(0))
        nsurv = jnp.minimum(nsurv, _CAP)
        nch = (nsurv + 15) // 16

        def sb(i, tlo):
            cand_lo = tlo | (jnp.int32(1) << (15 - i))
            cand = (t_hi << 16) | cand_lo

            def cb(ci, acc):
                v = surv_v[pl.ds(ci * 16, 16)]
                valid = (ci * 16 + lanes) < nsurv
                return acc + jnp.sum(jnp.where(valid & (v >= cand), 1, 0))
            cnt = lax.fori_loop(0, nch, cb, jnp.int32(0))
            return jnp.where(cnt >= k3, cand_lo, tlo)
        tlo = lax.fori_loop(0, 16, sb, jnp.int32(0))
        t = (t_hi << 16) | tlo

        def pm(i, c):
            v = keys_v[pl.ds(i * 16, 16)]
            out_v[pl.ds(i * 16, 16)] = jnp.where(v >= t, 1.0, 0.0)
            return c
        lax.fori_loop(0, nchunk, pm, jnp.int32(0))
        pltpu.sync_copy(out_v, out_hbm.at[row])
        return carry

    lax.fori_loop(0, _RPW, row_body, jnp.int32(0))


@jax.jit
def kernel(step, z_loga, eps):
    del step
    keys = _tc_keys(z_loga, eps)
    return _sc_select(keys)


# speedup vs baseline: 1.3779x; 1.3779x over previous
import functools

import jax
import jax.numpy as jnp
from jax import lax
from jax.experimental import pallas as pl
from jax.experimental.pallas import tpu as pltpu
from jax.experimental.pallas import tpu_sc as plsc

_ROWS = 128
_COLS = 32768
_K = 16384
_ROW_BLOCK = 16

_NC = 2
_NS = 16
_NW = _NC * _NS
_RPW = _ROWS // _NW
_NB = 256
_U = 8
_CAP1 = 8192
_CAP2 = 512


def _keys_body(z_ref, eps_ref, keys_ref):
    eps = jnp.clip(eps_ref[...], 1e-6, 1.0 - 1e-6)
    s = z_ref[...] - jnp.log(-jnp.log(eps))
    b = lax.bitcast_convert_type(s, jnp.int32)
    keys_ref[...] = b ^ ((b >> 31) & jnp.int32(0x7FFFFFFF))


def _tc_keys(z, eps):
    spec = pl.BlockSpec((_ROW_BLOCK, _COLS), lambda i: (i, 0))
    return pl.pallas_call(
        _keys_body,
        grid=(_ROWS // _ROW_BLOCK,),
        in_specs=[spec, spec],
        out_specs=spec,
        out_shape=jax.ShapeDtypeStruct((_ROWS, _COLS), jnp.int32),
    )(z, eps)


_mesh = plsc.VectorSubcoreMesh(core_axis_name="c", subcore_axis_name="s")


@functools.partial(
    pl.kernel,
    out_type=jax.ShapeDtypeStruct((_ROWS, _COLS), jnp.float32),
    mesh=_mesh,
    scratch_types=[
        pltpu.VMEM((_COLS,), jnp.int32),
        pltpu.VMEM((_COLS,), jnp.float32),
        pltpu.VMEM((_NB * _U,), jnp.int32),
        pltpu.VMEM((_NB * _U,), jnp.int32),
        pltpu.VMEM((_CAP1,), jnp.int32),
        pltpu.VMEM((_CAP2,), jnp.int32),
    ],
    compiler_params=pltpu.CompilerParams(needs_layout_passes=False),
)
def _sc_select(keys_hbm, out_hbm, keys_v, out_v, h1_v, h2_v, s1_v, s2_v):
    wid = lax.axis_index("s") * _NC + lax.axis_index("c")
    nouter = _COLS // (16 * _U)
    zeros16 = jnp.zeros((16,), jnp.int32)
    ones16 = jnp.ones((16,), jnp.int32)
    lanes = lax.iota(jnp.int32, 16)

    def scan_hist(h_ref, kk):
        def sbody(i, carry):
            cnt_ge, cnt_ab, tot = carry
            c = (_NB // 16 - 1) - i
            t16 = h_ref[pl.ds(c * 16, 16)]
            for u in range(1, _U):
                t16 = t16 + h_ref[pl.ds(u * _NB + c * 16, 16)]
            t_rev = lax.rev(t16, (0,))
            s_rev = plsc.cumsum(t_rev) + tot
            ge = s_rev >= kk
            cnt_ge = cnt_ge + jnp.sum(ge.astype(jnp.int32))
            cnt_ab = cnt_ab + jnp.sum(jnp.where(ge, 0, t_rev))
            tot = tot + jnp.sum(t16)
            return cnt_ge, cnt_ab, tot
        z = jnp.int32(0)
        cnt_ge, cnt_ab, _ = lax.fori_loop(0, _NB // 16, sbody, (z, z, z))
        return cnt_ge - 1, cnt_ab

    def row_body(r, carry):
        row = wid * _RPW + r
        pltpu.sync_copy(keys_hbm.at[row], keys_v)

        def zero_body(i, c):
            for u in range(_U):
                h1_v[pl.ds((i * _U + u) * 16, 16)] = zeros16
                h2_v[pl.ds((i * _U + u) * 16, 16)] = zeros16
            return c
        lax.fori_loop(0, _NB * _U // (16 * _U), zero_body, jnp.int32(0))

        def p1(i, c):
            base = i * (16 * _U)
            for u in range(_U):
                v = keys_v[pl.ds(base + u * 16, 16)]
                plsc.addupdate_scatter(
                    h1_v, [((v >> 24) + 128) + u * _NB], ones16)
            return c
        lax.fori_loop(0, nouter, p1, jnp.int32(0))
        b1, ca1 = scan_hist(h1_v, _K)
        k2 = _K - ca1

        def p2(i, cnt):
            base = i * (16 * _U)
            for u in range(_U):
                v = keys_v[pl.ds(base + u * 16, 16)]
                pred = ((v >> 24) + 128) == b1
                plsc.addupdate_scatter(
                    h2_v, [((v >> 16) & 0xFF) + u * _NB], ones16, mask=pred)
                off = jnp.minimum(cnt, _CAP1 - 16)
                plsc.store_compressed(s1_v.at[pl.ds(off, 16)], v, mask=pred)
                cnt = cnt + jnp.sum(pred.astype(jnp.int32))
            return cnt
        n1 = lax.fori_loop(0, nouter, p2, jnp.int32(0))
        n1 = jnp.minimum(n1, _CAP1)
        b2, ca2 = scan_hist(h2_v, k2)
        k3 = k2 - ca2
        t_hi = ((b1 - 128) << 8) | b2

        def pc(ci, cnt):
            v = s1_v[pl.ds(ci * 16, 16)]
            pred = ((v >> 16) == t_hi) & ((ci * 16 + lanes) < n1)
            off = jnp.minimum(cnt, _CAP2 - 16)
            plsc.store_compressed(s2_v.at[pl.ds(off, 16)], v, mask=pred)
            return cnt + jnp.sum(pred.astype(jnp.int32))
        n2 = lax.fori_loop(0, (n1 + 15) // 16, pc, jnp.int32(0))
        n2 = jnp.minimum(n2, _CAP2)
        nch2 = (n2 + 15) // 16

        def sb(i, tlo):
            cand_lo = tlo | (jnp.int32(1) << (15 - i))
            cand = (t_hi << 16) | cand_lo

            def cb(ci, acc):
                v = s2_v[pl.ds(ci * 16, 16)]
                valid = (ci * 16 + lanes) < n2
                return acc + jnp.sum(jnp.where(valid & (v >= cand), 1, 0))
            cnt = lax.fori_loop(0, nch2, cb, jnp.int32(0))
            return jnp.where(cnt >= k3, cand_lo, tlo)
        tlo = lax.fori_loop(0, 16, sb, jnp.int32(0))
        t = (t_hi << 16) | tlo

        def pm(i, c):
            base = i * (16 * _U)
            for u in range(_U):
                sl = pl.ds(base + u * 16, 16)
                out_v[sl] = jnp.where(keys_v[sl] >= t, 1.0, 0.0)
            return c
        lax.fori_loop(0, nouter, pm, jnp.int32(0))
        pltpu.sync_copy(out_v, out_hbm.at[row])
        return carry

    lax.fori_loop(0, _RPW, row_body, jnp.int32(0))


@jax.jit
def kernel(step, z_loga, eps):
    del step
    keys = _tc_keys(z_loga, eps)
    return _sc_select(keys)
